# baseline (speedup 1.0000x reference)
import jax
import jax.numpy as jnp
from jax import lax
from jax.experimental import pallas as pl
from jax.experimental.pallas import tpu as pltpu
from jax.experimental.pallas import tpu_sc as plsc

N = 320000
D = 128
S = 10000
S_PAD = 10240
DR = S_PAD // D

NC = 2
NS = 16
NW = NC * NS
L = 16

EPT = N // NW
CW = 80
NCH = EPT // CW
ZR = 16
Z1 = 1024
SPT = S_PAD // NS

K1_BLK = 32000
K1_GRID = N // K1_BLK

K3_BLK = 1024
K3_GRID = S_PAD // K3_BLK


def _k1_body(emb_ref, watt_ref, scores_ref, m_ref):
    i = pl.program_id(0)
    x = lax.dot_general(
        watt_ref[...], emb_ref[...],
        (((1,), (1,)), ((), ())),
        preferred_element_type=jnp.float32,
    )
    scores_ref[...] = x[None]
    bm = jnp.max(x)

    @pl.when(i == 0)
    def _():
        m_ref[...] = jnp.full((1, D), bm, jnp.float32)

    @pl.when(i > 0)
    def _():
        m_ref[...] = jnp.maximum(m_ref[...], bm)


def _k1(emb, W_att):
    return pl.pallas_call(
        _k1_body,
        grid=(K1_GRID,),
        in_specs=[
            pl.BlockSpec((K1_BLK, D), lambda i: (i, 0)),
            pl.BlockSpec((1, D), lambda i: (0, 0)),
        ],
        out_specs=[
            pl.BlockSpec((1, 1, K1_BLK), lambda i: (i, 0, 0)),
            pl.BlockSpec((1, D), lambda i: (0, 0)),
        ],
        out_shape=[
            jax.ShapeDtypeStruct((K1_GRID, 1, K1_BLK), jnp.float32),
            jax.ShapeDtypeStruct((1, D), jnp.float32),
        ],
    )(emb, W_att)


def _k2_body(emb, idx1, sco1, mrow, out_u, out_d,
             acc, acc_d,
             sb0, sb1, sb2, ix0, ix1, ix2, sc0, sc1, sc2,
             m_v, d_loc, zidx,
             sl0, sl1, sl2, ss0, ss1, ss2):
    sbs = (sb0, sb1, sb2)
    ixs = (ix0, ix1, ix2)
    scs = (sc0, sc1, sc2)
    sls = (sl0, sl1, sl2)
    sss = (ss0, ss1, ss2)

    cid = lax.axis_index("c")
    sid = lax.axis_index("s")
    g = cid * NS + sid
    base = g * EPT

    pltpu.sync_copy(mrow.at[0], m_v)
    m = m_v[pl.ds(0, L)][0]

    zrow = jnp.zeros((L,), jnp.float32)
    lane = lax.iota(jnp.int32, L)

    def _zero_sb(r, carry):
        for k in range(D // L):
            sb0[r, pl.ds(k * L, L)] = zrow
        return carry

    lax.fori_loop(0, CW, _zero_sb, 0)
    for j in range(SPT // CW):
        pltpu.sync_copy(sb0, acc.at[pl.ds(sid * SPT + j * CW, CW), :])

    @pl.when(sid == 0)
    def _():
        pltpu.sync_copy(sb0, acc_d)

    def _zero_d(r, carry):
        d_loc[pl.ds(r * L, L)] = zrow
        return carry

    lax.fori_loop(0, S_PAD // L, _zero_d, 0)

    def _mk_zidx(q, carry):
        zidx[pl.ds(q * L, L)] = q * L + lane
        return carry

    lax.fori_loop(0, DR // L, _mk_zidx, 0)

    plsc.subcore_barrier()

    def issue_loads(c, k):
        off = base + c * CW
        pltpu.async_copy(emb.at[pl.ds(off, CW), :], sbs[k], sls[k])
        pltpu.async_copy(idx1.at[pl.ds(off, CW)], ixs[k], sls[k])
        pltpu.async_copy(sco1.at[pl.ds(off, CW)], scs[k], sls[k])

    def wait_loads(c, k):
        off = base + c * CW
        pltpu.make_async_copy(emb.at[pl.ds(off, CW), :], sbs[k], sls[k]).wait()
        pltpu.make_async_copy(idx1.at[pl.ds(off, CW)], ixs[k], sls[k]).wait()
        pltpu.make_async_copy(sco1.at[pl.ds(off, CW)], scs[k], sls[k]).wait()

    def issue_scatter(k):
        pltpu.async_copy(sbs[k], acc.at[ixs[k]], sss[k], add=True)

    def wait_scatter(k):
        pltpu.make_async_copy(sbs[k], acc.at[ixs[k]], sss[k]).wait()

    def compute(k):
        sb_k, ix_k, sc_k = sbs[k], ixs[k], scs[k]

        def _grp(q, rcarry):
            ev = jnp.exp(sc_k[pl.ds(q * L, L)] - m)
            iv = ix_k[pl.ds(q * L, L)]
            plsc.addupdate_scatter(d_loc, [iv], ev)
            for j in range(L):
                r = q * L + j
                e_r = ev[j]
                for kk in range(D // L):
                    sb_k[r, pl.ds(kk * L, L)] = sb_k[r, pl.ds(kk * L, L)] * e_r
            return rcarry

        for q in range(CW // L):
            _grp(q, 0)

    issue_loads(0, 0)

    def _turn(c, k):
        @pl.when(jnp.logical_and(c >= 2, c < NCH + 2))
        def _():
            wait_scatter((k + 1) % 3)

        @pl.when(c + 1 < NCH)
        def _():
            issue_loads(c + 1, (k + 1) % 3)

        @pl.when(c < NCH)
        def _():
            wait_loads(c, k)
            compute(k)
            issue_scatter(k)

    NT = (NCH + 4 + 2) // 3

    def _turn3(t, carry):
        for k in range(3):
            _turn(3 * t + k, k)
        return carry

    lax.fori_loop(0, NT, _turn3, 0)

    def _pack_d(r, carry):
        for k in range(D // L):
            sb0[r, pl.ds(k * L, L)] = d_loc[pl.ds(r * D + k * L, L)]
        return carry

    lax.fori_loop(0, DR, _pack_d, 0)
    pltpu.sync_copy(sb0, acc_d.at[zidx], add=True)

    plsc.subcore_barrier()

    for j in range(SPT // CW):
        sl = pl.ds(sid * SPT + j * CW, CW)
        pltpu.sync_copy(acc.at[sl, :], out_u.at[cid, sl, :])

    @pl.when(sid == 0)
    def _():
        pltpu.sync_copy(acc_d, out_d.at[cid])


def _k2(emb, idx1, sco1, mrow):
    f = pl.kernel(
        _k2_body,
        out_type=[
            jax.ShapeDtypeStruct((NC, S_PAD, D), jnp.float32),
            jax.ShapeDtypeStruct((NC, DR, D), jnp.float32),
        ],
        mesh=plsc.VectorSubcoreMesh(core_axis_name="c", subcore_axis_name="s"),
        compiler_params=pltpu.CompilerParams(needs_layout_passes=False),
        scratch_types=[
            pltpu.VMEM_SHARED((S_PAD, D), jnp.float32),
            pltpu.VMEM_SHARED((DR, D), jnp.float32),
            pltpu.VMEM((CW, D), jnp.float32),
            pltpu.VMEM((CW, D), jnp.float32),
            pltpu.VMEM((CW, D), jnp.float32),
            pltpu.VMEM((CW,), jnp.int32),
            pltpu.VMEM((CW,), jnp.int32),
            pltpu.VMEM((CW,), jnp.int32),
            pltpu.VMEM((CW,), jnp.float32),
            pltpu.VMEM((CW,), jnp.float32),
            pltpu.VMEM((CW,), jnp.float32),
            pltpu.VMEM((D,), jnp.float32),
            pltpu.VMEM((S_PAD,), jnp.float32),
            pltpu.VMEM((DR,), jnp.int32),
            pltpu.SemaphoreType.DMA,
            pltpu.SemaphoreType.DMA,
            pltpu.SemaphoreType.DMA,
            pltpu.SemaphoreType.DMA,
            pltpu.SemaphoreType.DMA,
            pltpu.SemaphoreType.DMA,
        ],
    )
    return f(emb, idx1, sco1, mrow)


def _k3_body(u_ref, d_ref, w_ref, out_ref):
    u = u_ref[0] + u_ref[1]
    d = d_ref[0] + d_ref[1]
    r = jnp.where(d > 0.0, u / d, 0.0)
    out_ref[...] = lax.dot_general(
        r, w_ref[...], (((1,), (1,)), ((), ())),
        preferred_element_type=jnp.float32,
    )


def _k3(acc_u, acc_d, W_out):
    return pl.pallas_call(
        _k3_body,
        grid=(K3_GRID,),
        in_specs=[
            pl.BlockSpec((NC, K3_BLK, D), lambda i: (0, i, 0)),
            pl.BlockSpec((NC, K3_BLK, 1), lambda i: (0, i, 0)),
            pl.BlockSpec((D, D), lambda i: (0, 0)),
        ],
        out_specs=pl.BlockSpec((K3_BLK, D), lambda i: (i, 0)),
        out_shape=jax.ShapeDtypeStruct((S_PAD, D), jnp.float32),
    )(acc_u, acc_d, W_out)


def kernel(element_embeddings, element_to_sample_map, num_samples,
           W_att, W_out):
    idx1 = element_to_sample_map.astype(jnp.int32)
    scores, mrow = _k1(element_embeddings, W_att)
    acc_u, acc_d = _k2(element_embeddings, idx1, scores.reshape(N), mrow)
    d_col = acc_d.reshape(NC, S_PAD, 1)
    return _k3(acc_u, d_col, W_out)[:S]


# speedup vs baseline: 1.0057x; 1.0057x over previous
import jax
import jax.numpy as jnp
from jax import lax
from jax.experimental import pallas as pl
from jax.experimental.pallas import tpu as pltpu
from jax.experimental.pallas import tpu_sc as plsc

N = 320000
D = 128
S = 10000
S_PAD = 10240
DR = S_PAD // D

NC = 2
NS = 16
NW = NC * NS
L = 16

EPT = N // NW
CW = 80
NCH = EPT // CW
ZR = 16
Z1 = 1024
SPT = S_PAD // NS

K1_BLK = 16000
K1_GRID = N // K1_BLK

K3_BLK = 1024
K3_GRID = S_PAD // K3_BLK


def _k1_body(emb_ref, watt_ref, scores_ref, m_ref):
    i = pl.program_id(0)
    x = lax.dot_general(
        watt_ref[...], emb_ref[...],
        (((1,), (1,)), ((), ())),
        preferred_element_type=jnp.float32,
    )
    scores_ref[...] = x[None]
    bm = jnp.max(x)

    @pl.when(i == 0)
    def _():
        m_ref[...] = jnp.full((1, D), bm, jnp.float32)

    @pl.when(i > 0)
    def _():
        m_ref[...] = jnp.maximum(m_ref[...], bm)


def _k1(emb, W_att):
    return pl.pallas_call(
        _k1_body,
        grid=(K1_GRID,),
        in_specs=[
            pl.BlockSpec((K1_BLK, D), lambda i: (i, 0)),
            pl.BlockSpec((1, D), lambda i: (0, 0)),
        ],
        out_specs=[
            pl.BlockSpec((1, 1, K1_BLK), lambda i: (i, 0, 0)),
            pl.BlockSpec((1, D), lambda i: (0, 0)),
        ],
        out_shape=[
            jax.ShapeDtypeStruct((K1_GRID, 1, K1_BLK), jnp.float32),
            jax.ShapeDtypeStruct((1, D), jnp.float32),
        ],
    )(emb, W_att)


def _k2_body(emb, idx1, sco1, mrow, out_u, out_d,
             acc, acc_d,
             sb0, sb1, sb2, ix0, ix1, ix2, sc0, sc1, sc2,
             m_v, d_loc, zidx,
             sl0, sl1, sl2, ss0, ss1, ss2):
    sbs = (sb0, sb1, sb2)
    ixs = (ix0, ix1, ix2)
    scs = (sc0, sc1, sc2)
    sls = (sl0, sl1, sl2)
    sss = (ss0, ss1, ss2)

    cid = lax.axis_index("c")
    sid = lax.axis_index("s")
    g = cid * NS + sid
    base = g * EPT

    pltpu.sync_copy(mrow.at[0], m_v)
    m = m_v[pl.ds(0, L)][0]

    zrow = jnp.zeros((L,), jnp.float32)
    lane = lax.iota(jnp.int32, L)

    def _zero_sb(r, carry):
        for k in range(D // L):
            sb0[r, pl.ds(k * L, L)] = zrow
        return carry

    lax.fori_loop(0, CW, _zero_sb, 0)
    for j in range(SPT // CW):
        pltpu.sync_copy(sb0, acc.at[pl.ds(sid * SPT + j * CW, CW), :])

    @pl.when(sid == 0)
    def _():
        pltpu.sync_copy(sb0, acc_d)

    def _zero_d(r, carry):
        d_loc[pl.ds(r * L, L)] = zrow
        return carry

    lax.fori_loop(0, S_PAD // L, _zero_d, 0)

    def _mk_zidx(q, carry):
        zidx[pl.ds(q * L, L)] = q * L + lane
        return carry

    lax.fori_loop(0, DR // L, _mk_zidx, 0)

    plsc.subcore_barrier()

    def issue_loads(c, k):
        off = base + c * CW
        pltpu.async_copy(emb.at[pl.ds(off, CW), :], sbs[k], sls[k])
        pltpu.async_copy(idx1.at[pl.ds(off, CW)], ixs[k], sls[k])
        pltpu.async_copy(sco1.at[pl.ds(off, CW)], scs[k], sls[k])

    def wait_loads(c, k):
        off = base + c * CW
        pltpu.make_async_copy(emb.at[pl.ds(off, CW), :], sbs[k], sls[k]).wait()
        pltpu.make_async_copy(idx1.at[pl.ds(off, CW)], ixs[k], sls[k]).wait()
        pltpu.make_async_copy(sco1.at[pl.ds(off, CW)], scs[k], sls[k]).wait()

    def issue_scatter(k):
        pltpu.async_copy(sbs[k], acc.at[ixs[k]], sss[k], add=True)

    def wait_scatter(k):
        pltpu.make_async_copy(sbs[k], acc.at[ixs[k]], sss[k]).wait()

    def compute(k):
        sb_k, ix_k, sc_k = sbs[k], ixs[k], scs[k]

        def _grp(q, rcarry):
            ev = jnp.exp(sc_k[pl.ds(q * L, L)] - m)
            iv = ix_k[pl.ds(q * L, L)]
            plsc.addupdate_scatter(d_loc, [iv], ev)
            for j in range(L):
                r = q * L + j
                e_r = ev[j]
                for kk in range(D // L):
                    sb_k[r, pl.ds(kk * L, L)] = sb_k[r, pl.ds(kk * L, L)] * e_r
            return rcarry

        for q in range(CW // L):
            _grp(q, 0)

    issue_loads(0, 0)

    def _turn(c, k):
        @pl.when(jnp.logical_and(c >= 2, c < NCH + 2))
        def _():
            wait_scatter((k + 1) % 3)

        @pl.when(c + 1 < NCH)
        def _():
            issue_loads(c + 1, (k + 1) % 3)

        @pl.when(c < NCH)
        def _():
            wait_loads(c, k)
            compute(k)
            issue_scatter(k)

    NT = (NCH + 4 + 2) // 3

    def _turn3(t, carry):
        for k in range(3):
            _turn(3 * t + k, k)
        return carry

    lax.fori_loop(0, NT, _turn3, 0)

    def _pack_d(r, carry):
        for k in range(D // L):
            sb0[r, pl.ds(k * L, L)] = d_loc[pl.ds(r * D + k * L, L)]
        return carry

    lax.fori_loop(0, DR, _pack_d, 0)
    pltpu.sync_copy(sb0, acc_d.at[zidx], add=True)

    plsc.subcore_barrier()

    for j in range(SPT // CW):
        sl = pl.ds(sid * SPT + j * CW, CW)
        pltpu.sync_copy(acc.at[sl, :], out_u.at[cid, sl, :])

    @pl.when(sid == 0)
    def _():
        pltpu.sync_copy(acc_d, out_d.at[cid])


def _k2(emb, idx1, sco1, mrow):
    f = pl.kernel(
        _k2_body,
        out_type=[
            jax.ShapeDtypeStruct((NC, S_PAD, D), jnp.float32),
            jax.ShapeDtypeStruct((NC, DR, D), jnp.float32),
        ],
        mesh=plsc.VectorSubcoreMesh(core_axis_name="c", subcore_axis_name="s"),
        compiler_params=pltpu.CompilerParams(needs_layout_passes=False),
        scratch_types=[
            pltpu.VMEM_SHARED((S_PAD, D), jnp.float32),
            pltpu.VMEM_SHARED((DR, D), jnp.float32),
            pltpu.VMEM((CW, D), jnp.float32),
            pltpu.VMEM((CW, D), jnp.float32),
            pltpu.VMEM((CW, D), jnp.float32),
            pltpu.VMEM((CW,), jnp.int32),
            pltpu.VMEM((CW,), jnp.int32),
            pltpu.VMEM((CW,), jnp.int32),
            pltpu.VMEM((CW,), jnp.float32),
            pltpu.VMEM((CW,), jnp.float32),
            pltpu.VMEM((CW,), jnp.float32),
            pltpu.VMEM((D,), jnp.float32),
            pltpu.VMEM((S_PAD,), jnp.float32),
            pltpu.VMEM((DR,), jnp.int32),
            pltpu.SemaphoreType.DMA,
            pltpu.SemaphoreType.DMA,
            pltpu.SemaphoreType.DMA,
            pltpu.SemaphoreType.DMA,
            pltpu.SemaphoreType.DMA,
            pltpu.SemaphoreType.DMA,
        ],
    )
    return f(emb, idx1, sco1, mrow)


def _k3_body(u_ref, d_ref, w_ref, out_ref):
    u = u_ref[0] + u_ref[1]
    d = d_ref[0] + d_ref[1]
    r = jnp.where(d > 0.0, u / d, 0.0)
    out_ref[...] = lax.dot_general(
        r, w_ref[...], (((1,), (1,)), ((), ())),
        preferred_element_type=jnp.float32,
    )


def _k3(acc_u, acc_d, W_out):
    return pl.pallas_call(
        _k3_body,
        grid=(K3_GRID,),
        in_specs=[
            pl.BlockSpec((NC, K3_BLK, D), lambda i: (0, i, 0)),
            pl.BlockSpec((NC, K3_BLK, 1), lambda i: (0, i, 0)),
            pl.BlockSpec((D, D), lambda i: (0, 0)),
        ],
        out_specs=pl.BlockSpec((K3_BLK, D), lambda i: (i, 0)),
        out_shape=jax.ShapeDtypeStruct((S_PAD, D), jnp.float32),
    )(acc_u, acc_d, W_out)


def kernel(element_embeddings, element_to_sample_map, num_samples,
           W_att, W_out):
    idx1 = element_to_sample_map.astype(jnp.int32)
    scores, mrow = _k1(element_embeddings, W_att)
    acc_u, acc_d = _k2(element_embeddings, idx1, scores.reshape(N), mrow)
    d_col = acc_d.reshape(NC, S_PAD, 1)
    return _k3(acc_u, d_col, W_out)[:S]


# speedup vs baseline: 1.1743x; 1.1676x over previous
import jax
import jax.numpy as jnp
from jax import lax
from jax.experimental import pallas as pl
from jax.experimental.pallas import tpu as pltpu
from jax.experimental.pallas import tpu_sc as plsc

N = 320000
D = 128
S = 10000
S_PAD = 10240
DR = S_PAD // D

NC = 2
NS = 16
NW = NC * NS
L = 16

EPT = N // NW
CW = 80
NCH = EPT // CW
ZR = 16
Z1 = 1024
SPT = S_PAD // NS

K1_BLK = 16000
K1_GRID = N // K1_BLK

K3_BLK = 1024
K3_GRID = S_PAD // K3_BLK


def _k1_body(emb_ref, watt_ref, scores_ref, m_ref):
    i = pl.program_id(0)
    x = lax.dot_general(
        watt_ref[...], emb_ref[...],
        (((1,), (1,)), ((), ())),
        preferred_element_type=jnp.float32,
    )
    scores_ref[...] = x[None]
    bm = jnp.max(x)

    @pl.when(i == 0)
    def _():
        m_ref[...] = jnp.full((1, D), bm, jnp.float32)

    @pl.when(i > 0)
    def _():
        m_ref[...] = jnp.maximum(m_ref[...], bm)


def _k1(emb, W_att):
    return pl.pallas_call(
        _k1_body,
        grid=(K1_GRID,),
        in_specs=[
            pl.BlockSpec((K1_BLK, D), lambda i: (i, 0)),
            pl.BlockSpec((1, D), lambda i: (0, 0)),
        ],
        out_specs=[
            pl.BlockSpec((1, 1, K1_BLK), lambda i: (i, 0, 0)),
            pl.BlockSpec((1, D), lambda i: (0, 0)),
        ],
        out_shape=[
            jax.ShapeDtypeStruct((K1_GRID, 1, K1_BLK), jnp.float32),
            jax.ShapeDtypeStruct((1, D), jnp.float32),
        ],
    )(emb, W_att)


def _k2_body(emb, idx1, sco1, mrow, out_u, out_d,
             acc, acc_d,
             sb0, sb1, sb2, ix0, ix1, ix2, sc0, sc1, sc2,
             m_v, d_loc, zidx,
             sl0, sl1, sl2, ss0, ss1, ss2):
    sbs = (sb0, sb1, sb2)
    ixs = (ix0, ix1, ix2)
    scs = (sc0, sc1, sc2)
    sls = (sl0, sl1, sl2)
    sss = (ss0, ss1, ss2)

    cid = lax.axis_index("c")
    sid = lax.axis_index("s")
    g = cid * NS + sid
    base = g * EPT

    pltpu.sync_copy(mrow.at[0], m_v)
    m = m_v[pl.ds(0, L)][0]

    zrow = jnp.zeros((L,), jnp.float32)
    lane = lax.iota(jnp.int32, L)

    def _zero_sb(r, carry):
        for k in range(D // L):
            sb0[r, pl.ds(k * L, L)] = zrow
        return carry

    lax.fori_loop(0, CW, _zero_sb, 0)
    for j in range(SPT // CW):
        pltpu.sync_copy(sb0, acc.at[pl.ds(sid * SPT + j * CW, CW), :])

    @pl.when(sid == 0)
    def _():
        pltpu.sync_copy(sb0, acc_d)

    def _zero_d(r, carry):
        d_loc[pl.ds(r * L, L)] = zrow
        return carry

    lax.fori_loop(0, S_PAD // L, _zero_d, 0)

    def _mk_zidx(q, carry):
        zidx[pl.ds(q * L, L)] = q * L + lane
        return carry

    lax.fori_loop(0, DR // L, _mk_zidx, 0)

    plsc.subcore_barrier()

    def issue_loads(c, k):
        off = base + c * CW
        pltpu.async_copy(emb.at[pl.ds(off, CW), :], sbs[k], sls[k])
        pltpu.async_copy(idx1.at[pl.ds(off, CW)], ixs[k], sls[k])
        pltpu.async_copy(sco1.at[pl.ds(off, CW)], scs[k], sls[k])

    def wait_loads(c, k):
        off = base + c * CW
        pltpu.make_async_copy(emb.at[pl.ds(off, CW), :], sbs[k], sls[k]).wait()
        pltpu.make_async_copy(idx1.at[pl.ds(off, CW)], ixs[k], sls[k]).wait()
        pltpu.make_async_copy(sco1.at[pl.ds(off, CW)], scs[k], sls[k]).wait()

    def issue_scatter(k):
        pltpu.async_copy(sbs[k], acc.at[ixs[k]], sss[k], add=True)

    def wait_scatter(k):
        pltpu.make_async_copy(sbs[k], acc.at[ixs[k]], sss[k]).wait()

    def compute(k):
        sb_k, ix_k, sc_k = sbs[k], ixs[k], scs[k]

        def _grp(q, rcarry):
            ev = jnp.exp(sc_k[pl.ds(q * L, L)] - m)
            iv = ix_k[pl.ds(q * L, L)]
            plsc.addupdate_scatter(d_loc, [iv], ev)
            for j in range(L):
                r = q * L + j
                e_r = ev[j]
                for kk in range(D // L):
                    sb_k[r, pl.ds(kk * L, L)] = sb_k[r, pl.ds(kk * L, L)] * e_r
            return rcarry

        lax.fori_loop(0, CW // L, _grp, 0)

    issue_loads(0, 0)

    def _turn(c, k):
        @pl.when(jnp.logical_and(c >= 2, c < NCH + 2))
        def _():
            wait_scatter((k + 1) % 3)

        @pl.when(c + 1 < NCH)
        def _():
            issue_loads(c + 1, (k + 1) % 3)

        @pl.when(c < NCH)
        def _():
            wait_loads(c, k)
            compute(k)
            issue_scatter(k)

    NT = (NCH + 4 + 2) // 3

    def _turn3(t, carry):
        for k in range(3):
            _turn(3 * t + k, k)
        return carry

    lax.fori_loop(0, NT, _turn3, 0)

    def _pack_d(r, carry):
        for k in range(D // L):
            sb0[r, pl.ds(k * L, L)] = d_loc[pl.ds(r * D + k * L, L)]
        return carry

    lax.fori_loop(0, DR, _pack_d, 0)
    pltpu.sync_copy(sb0, acc_d.at[zidx], add=True)

    plsc.subcore_barrier()

    for j in range(SPT // CW):
        sl = pl.ds(sid * SPT + j * CW, CW)
        pltpu.sync_copy(acc.at[sl, :], out_u.at[cid, sl, :])

    @pl.when(sid == 0)
    def _():
        pltpu.sync_copy(acc_d, out_d.at[cid])


def _k2(emb, idx1, sco1, mrow):
    f = pl.kernel(
        _k2_body,
        out_type=[
            jax.ShapeDtypeStruct((NC, S_PAD, D), jnp.float32),
            jax.ShapeDtypeStruct((NC, DR, D), jnp.float32),
        ],
        mesh=plsc.VectorSubcoreMesh(core_axis_name="c", subcore_axis_name="s"),
        compiler_params=pltpu.CompilerParams(needs_layout_passes=False),
        scratch_types=[
            pltpu.VMEM_SHARED((S_PAD, D), jnp.float32),
            pltpu.VMEM_SHARED((DR, D), jnp.float32),
            pltpu.VMEM((CW, D), jnp.float32),
            pltpu.VMEM((CW, D), jnp.float32),
            pltpu.VMEM((CW, D), jnp.float32),
            pltpu.VMEM((CW,), jnp.int32),
            pltpu.VMEM((CW,), jnp.int32),
            pltpu.VMEM((CW,), jnp.int32),
            pltpu.VMEM((CW,), jnp.float32),
            pltpu.VMEM((CW,), jnp.float32),
            pltpu.VMEM((CW,), jnp.float32),
            pltpu.VMEM((D,), jnp.float32),
            pltpu.VMEM((S_PAD,), jnp.float32),
            pltpu.VMEM((DR,), jnp.int32),
            pltpu.SemaphoreType.DMA,
            pltpu.SemaphoreType.DMA,
            pltpu.SemaphoreType.DMA,
            pltpu.SemaphoreType.DMA,
            pltpu.SemaphoreType.DMA,
            pltpu.SemaphoreType.DMA,
        ],
    )
    return f(emb, idx1, sco1, mrow)


def _k3_body(u_ref, d_ref, w_ref, out_ref):
    u = u_ref[0] + u_ref[1]
    d = d_ref[0] + d_ref[1]
    r = jnp.where(d > 0.0, u / d, 0.0)
    out_ref[...] = lax.dot_general(
        r, w_ref[...], (((1,), (1,)), ((), ())),
        preferred_element_type=jnp.float32,
    )


def _k3(acc_u, acc_d, W_out):
    return pl.pallas_call(
        _k3_body,
        grid=(K3_GRID,),
        in_specs=[
            pl.BlockSpec((NC, K3_BLK, D), lambda i: (0, i, 0)),
            pl.BlockSpec((NC, K3_BLK, 1), lambda i: (0, i, 0)),
            pl.BlockSpec((D, D), lambda i: (0, 0)),
        ],
        out_specs=pl.BlockSpec((K3_BLK, D), lambda i: (i, 0)),
        out_shape=jax.ShapeDtypeStruct((S_PAD, D), jnp.float32),
    )(acc_u, acc_d, W_out)


def kernel(element_embeddings, element_to_sample_map, num_samples,
           W_att, W_out):
    idx1 = element_to_sample_map.astype(jnp.int32)
    scores, mrow = _k1(element_embeddings, W_att)
    acc_u, acc_d = _k2(element_embeddings, idx1, scores.reshape(N), mrow)
    d_col = acc_d.reshape(NC, S_PAD, 1)
    return _k3(acc_u, d_col, W_out)[:S]


# speedup vs baseline: 1.1999x; 1.0218x over previous
import jax
import jax.numpy as jnp
from jax import lax
from jax.experimental import pallas as pl
from jax.experimental.pallas import tpu as pltpu
from jax.experimental.pallas import tpu_sc as plsc

N = 320000
D = 128
S = 10000
S_PAD = 10240
DR = S_PAD // D

NC = 2
NS = 16
NW = NC * NS
L = 16

EPT = N // NW
CW = 80
NCH = EPT // CW
ZR = 16
Z1 = 1024
SPT = S_PAD // NS

K1_BLK = 20000
K1_GRID = N // K1_BLK

K3_BLK = 1000
K3_GRID = S // K3_BLK


def _k1_body(emb_ref, watt_ref, scores_ref, m_ref):
    i = pl.program_id(0)
    x = lax.dot_general(
        watt_ref[...], emb_ref[...],
        (((1,), (1,)), ((), ())),
        preferred_element_type=jnp.float32,
    )
    scores_ref[...] = x[None]
    bm = jnp.max(x)

    @pl.when(i == 0)
    def _():
        m_ref[...] = jnp.full((1, D), bm, jnp.float32)

    @pl.when(i > 0)
    def _():
        m_ref[...] = jnp.maximum(m_ref[...], bm)


def _k1(emb, W_att):
    return pl.pallas_call(
        _k1_body,
        grid=(K1_GRID,),
        in_specs=[
            pl.BlockSpec((K1_BLK, D), lambda i: (i, 0)),
            pl.BlockSpec((1, D), lambda i: (0, 0)),
        ],
        out_specs=[
            pl.BlockSpec((1, 1, K1_BLK), lambda i: (i, 0, 0)),
            pl.BlockSpec((1, D), lambda i: (0, 0)),
        ],
        out_shape=[
            jax.ShapeDtypeStruct((K1_GRID, 1, K1_BLK), jnp.float32),
            jax.ShapeDtypeStruct((1, D), jnp.float32),
        ],
    )(emb, W_att)


def _k2_body(emb, idx1, sco1, mrow, out_u, out_d,
             acc, acc_d,
             sb0, sb1, sb2, ix0, ix1, ix2, sc0, sc1, sc2,
             m_v, d_loc, zidx,
             sl0, sl1, sl2, ss0, ss1, ss2):
    sbs = (sb0, sb1, sb2)
    ixs = (ix0, ix1, ix2)
    scs = (sc0, sc1, sc2)
    sls = (sl0, sl1, sl2)
    sss = (ss0, ss1, ss2)

    cid = lax.axis_index("c")
    sid = lax.axis_index("s")
    g = cid * NS + sid
    base = g * EPT

    pltpu.sync_copy(mrow.at[0], m_v)
    m = m_v[pl.ds(0, L)][0]

    zrow = jnp.zeros((L,), jnp.float32)
    lane = lax.iota(jnp.int32, L)

    def _zero_sb(r, carry):
        for k in range(D // L):
            sb0[r, pl.ds(k * L, L)] = zrow
        return carry

    lax.fori_loop(0, CW, _zero_sb, 0)
    for j in range(SPT // CW):
        pltpu.sync_copy(sb0, acc.at[pl.ds(sid * SPT + j * CW, CW), :])

    @pl.when(sid == 0)
    def _():
        pltpu.sync_copy(sb0, acc_d)

    def _zero_d(r, carry):
        d_loc[pl.ds(r * L, L)] = zrow
        return carry

    lax.fori_loop(0, S_PAD // L, _zero_d, 0)

    def _mk_zidx(q, carry):
        zidx[pl.ds(q * L, L)] = q * L + lane
        return carry

    lax.fori_loop(0, DR // L, _mk_zidx, 0)

    plsc.subcore_barrier()

    def issue_loads(c, k):
        off = base + c * CW
        pltpu.async_copy(emb.at[pl.ds(off, CW), :], sbs[k], sls[k])
        pltpu.async_copy(idx1.at[pl.ds(off, CW)], ixs[k], sls[k])
        pltpu.async_copy(sco1.at[pl.ds(off, CW)], scs[k], sls[k])

    def wait_loads(c, k):
        off = base + c * CW
        pltpu.make_async_copy(emb.at[pl.ds(off, CW), :], sbs[k], sls[k]).wait()
        pltpu.make_async_copy(idx1.at[pl.ds(off, CW)], ixs[k], sls[k]).wait()
        pltpu.make_async_copy(sco1.at[pl.ds(off, CW)], scs[k], sls[k]).wait()

    def issue_scatter(k):
        pltpu.async_copy(sbs[k], acc.at[ixs[k]], sss[k], add=True)

    def wait_scatter(k):
        pltpu.make_async_copy(sbs[k], acc.at[ixs[k]], sss[k]).wait()

    def compute(k):
        sb_k, ix_k, sc_k = sbs[k], ixs[k], scs[k]

        def _grp(q, rcarry):
            ev = jnp.exp(sc_k[pl.ds(q * L, L)] - m)
            iv = ix_k[pl.ds(q * L, L)]
            plsc.addupdate_scatter(d_loc, [iv], ev)
            for j in range(L):
                r = q * L + j
                e_r = ev[j]
                for kk in range(D // L):
                    sb_k[r, pl.ds(kk * L, L)] = sb_k[r, pl.ds(kk * L, L)] * e_r
            return rcarry

        lax.fori_loop(0, CW // L, _grp, 0)

    issue_loads(0, 0)

    def _turn(c, k):
        @pl.when(jnp.logical_and(c >= 2, c < NCH + 2))
        def _():
            wait_scatter((k + 1) % 3)

        @pl.when(c + 1 < NCH)
        def _():
            issue_loads(c + 1, (k + 1) % 3)

        @pl.when(c < NCH)
        def _():
            wait_loads(c, k)
            compute(k)
            issue_scatter(k)

    NT = (NCH + 4 + 2) // 3

    def _turn3(t, carry):
        for k in range(3):
            _turn(3 * t + k, k)
        return carry

    lax.fori_loop(0, NT, _turn3, 0)

    def _pack_d(r, carry):
        for k in range(D // L):
            sb0[r, pl.ds(k * L, L)] = d_loc[pl.ds(r * D + k * L, L)]
        return carry

    lax.fori_loop(0, DR, _pack_d, 0)
    pltpu.sync_copy(sb0, acc_d.at[zidx], add=True)

    plsc.subcore_barrier()

    for j in range(SPT // CW):
        sl = pl.ds(sid * SPT + j * CW, CW)
        pltpu.sync_copy(acc.at[sl, :], out_u.at[cid, sl, :])

    @pl.when(sid == 0)
    def _():
        pltpu.sync_copy(acc_d, out_d.at[cid])


def _k2(emb, idx1, sco1, mrow):
    f = pl.kernel(
        _k2_body,
        out_type=[
            jax.ShapeDtypeStruct((NC, S_PAD, D), jnp.float32),
            jax.ShapeDtypeStruct((NC, DR, D), jnp.float32),
        ],
        mesh=plsc.VectorSubcoreMesh(core_axis_name="c", subcore_axis_name="s"),
        compiler_params=pltpu.CompilerParams(needs_layout_passes=False),
        scratch_types=[
            pltpu.VMEM_SHARED((S_PAD, D), jnp.float32),
            pltpu.VMEM_SHARED((DR, D), jnp.float32),
            pltpu.VMEM((CW, D), jnp.float32),
            pltpu.VMEM((CW, D), jnp.float32),
            pltpu.VMEM((CW, D), jnp.float32),
            pltpu.VMEM((CW,), jnp.int32),
            pltpu.VMEM((CW,), jnp.int32),
            pltpu.VMEM((CW,), jnp.int32),
            pltpu.VMEM((CW,), jnp.float32),
            pltpu.VMEM((CW,), jnp.float32),
            pltpu.VMEM((CW,), jnp.float32),
            pltpu.VMEM((D,), jnp.float32),
            pltpu.VMEM((S_PAD,), jnp.float32),
            pltpu.VMEM((DR,), jnp.int32),
            pltpu.SemaphoreType.DMA,
            pltpu.SemaphoreType.DMA,
            pltpu.SemaphoreType.DMA,
            pltpu.SemaphoreType.DMA,
            pltpu.SemaphoreType.DMA,
            pltpu.SemaphoreType.DMA,
        ],
    )
    return f(emb, idx1, sco1, mrow)


def _k3_body(u_ref, d_ref, w_ref, out_ref):
    u = u_ref[0] + u_ref[1]
    d = d_ref[0] + d_ref[1]
    r = jnp.where(d > 0.0, u / d, 0.0)
    out_ref[...] = lax.dot_general(
        r, w_ref[...], (((1,), (1,)), ((), ())),
        preferred_element_type=jnp.float32,
    )


def _k3(acc_u, acc_d, W_out):
    return pl.pallas_call(
        _k3_body,
        grid=(K3_GRID,),
        in_specs=[
            pl.BlockSpec((NC, K3_BLK, D), lambda i: (0, i, 0)),
            pl.BlockSpec((NC, K3_BLK, 1), lambda i: (0, i, 0)),
            pl.BlockSpec((D, D), lambda i: (0, 0)),
        ],
        out_specs=pl.BlockSpec((K3_BLK, D), lambda i: (i, 0)),
        out_shape=jax.ShapeDtypeStruct((S_PAD, D), jnp.float32),
    )(acc_u, acc_d, W_out)


def kernel(element_embeddings, element_to_sample_map, num_samples,
           W_att, W_out):
    idx1 = element_to_sample_map.astype(jnp.int32)
    scores, mrow = _k1(element_embeddings, W_att)
    acc_u, acc_d = _k2(element_embeddings, idx1, scores.reshape(N), mrow)
    d_col = acc_d.reshape(NC, S_PAD, 1)
    return _k3(acc_u, d_col, W_out)
